# Initial kernel scaffold; baseline (speedup 1.0000x reference)
import functools

import jax
import jax.numpy as jnp
from jax import lax
from jax.experimental import pallas as pl
from jax.experimental.pallas import tpu as pltpu
from jax.experimental.pallas import tpu_sc as plsc

_B = 64
_CAND = 32
_SW = 64
_C = 100
_K = 10
_R = _B * _CAND * _SW
_NC = 2
_NS = 16
_NW = _NC * _NS
_RPW = _R // _NW
_CHUNK = 128
_NCHUNK = _RPW // _CHUNK
_G = 16
_GPC = _CHUNK // _G
_RPB = _CAND * _SW
_CPB = _RPB // _CHUNK


def _insert(ms, x):
    out = [jnp.maximum(x, ms[0])]
    for i in range(1, len(ms)):
        out.append(jnp.maximum(jnp.minimum(x, ms[i - 1]), ms[i]))
    return tuple(out)


@functools.partial(
    pl.kernel,
    out_type=jax.ShapeDtypeStruct((_R * _C,), jnp.float32),
    mesh=plsc.VectorSubcoreMesh(core_axis_name="c", subcore_axis_name="s"),
    scratch_types=[
        pltpu.VMEM((_CHUNK * _C,), jnp.float32),
        pltpu.VMEM((_CHUNK * _C,), jnp.float32),
        pltpu.VMEM((_G * _C,), jnp.float32),
        pltpu.VMEM((2, _C), jnp.float32),
    ],
)
def _topk_mask_norm(a_hbm, mask_hbm, out_hbm, inbuf, outbuf, tbuf, mask_v):
    wid = lax.axis_index("s") * _NC + lax.axis_index("c")
    row0 = wid * _RPW
    pltpu.sync_copy(mask_hbm.at[pl.ds(2 * wid, 2)], mask_v)
    iota = lax.iota(jnp.int32, 16) * _C
    neg_inf = jnp.full((16,), -jnp.inf, jnp.float32)
    zero = jnp.zeros((16,), jnp.float32)

    def chunk_body(ci, carry):
        crow = row0 + ci * _CHUNK
        pltpu.sync_copy(a_hbm.at[pl.ds(crow * _C, _CHUNK * _C)], inbuf)
        b_local = ci // _CPB

        def group_body(g, carry2):
            gbase = g * (_G * _C)

            def p1(j, ms):
                x = plsc.load_gather(inbuf, [iota + (gbase + j)])
                tbuf[pl.ds(j * _G, _G)] = x
                return _insert(ms, x)

            ms = lax.fori_loop(0, _C, p1, (neg_inf,) * _K)
            m0, thr = ms[0], ms[_K - 1]

            def p2(j, zs):
                z, s = zs
                x = tbuf[pl.ds(j * _G, _G)]
                e = jnp.where(x >= thr, jnp.exp(x - m0), 0.0)
                t = e * mask_v[b_local, j]
                tbuf[pl.ds(j * _G, _G)] = t
                return (z + e, s + t)

            z, s = lax.fori_loop(0, _C, p2, (zero, zero))
            inv = 1.0 / (s + 1e-8 * z)

            def p3(j, c3):
                t = tbuf[pl.ds(j * _G, _G)]
                plsc.store_scatter(outbuf, [iota + (gbase + j)], t * inv)
                return c3

            lax.fori_loop(0, _C, p3, 0)
            return carry2

        lax.fori_loop(0, _GPC, group_body, 0)
        pltpu.sync_copy(outbuf, out_hbm.at[pl.ds(crow * _C, _CHUNK * _C)])
        return carry

    lax.fori_loop(0, _NCHUNK, chunk_body, 0)


def kernel(a, mask):
    a2 = a.reshape(_R * _C)
    out = _topk_mask_norm(a2, mask)
    return out.reshape(_B, _CAND, _SW, _C)


# speedup vs baseline: 5.7717x; 5.7717x over previous
import functools

import jax
import jax.numpy as jnp
from jax import lax
from jax.experimental import pallas as pl
from jax.experimental.pallas import tpu as pltpu
from jax.experimental.pallas import tpu_sc as plsc

_B = 64
_CAND = 32
_SW = 64
_C = 100
_K = 10
_R = _B * _CAND * _SW
_NC = 2
_NS = 16
_NW = _NC * _NS
_RPW = _R // _NW
_CHUNK = 128
_NCHUNK = _RPW // _CHUNK
_G = 16
_GPC = _CHUNK // _G
_RPB = _CAND * _SW
_CPB = _RPB // _CHUNK


def _insert(ms, x):
    out = [jnp.maximum(x, ms[0])]
    for i in range(1, len(ms)):
        out.append(jnp.maximum(jnp.minimum(x, ms[i - 1]), ms[i]))
    return tuple(out)


@functools.partial(
    pl.kernel,
    out_type=jax.ShapeDtypeStruct((_R * _C,), jnp.float32),
    mesh=plsc.VectorSubcoreMesh(core_axis_name="c", subcore_axis_name="s"),
    compiler_params=pltpu.CompilerParams(needs_layout_passes=False),
    scratch_types=[
        pltpu.VMEM((_CHUNK * _C,), jnp.float32),
        pltpu.VMEM((_CHUNK * _C,), jnp.float32),
        pltpu.VMEM((_G * _C,), jnp.float32),
        pltpu.VMEM((2, _C), jnp.float32),
    ],
)
def _topk_mask_norm(a_hbm, mask_hbm, out_hbm, inbuf, outbuf, tbuf, mask_v):
    wid = lax.axis_index("s") * _NC + lax.axis_index("c")
    row0 = wid * _RPW
    pltpu.sync_copy(mask_hbm.at[pl.ds(2 * wid, 2)], mask_v)
    iota = lax.iota(jnp.int32, 16) * _C
    neg_inf = jnp.full((16,), -jnp.inf, jnp.float32)
    zero = jnp.zeros((16,), jnp.float32)

    def chunk_body(ci, carry):
        crow = row0 + ci * _CHUNK
        pltpu.sync_copy(a_hbm.at[pl.ds(crow * _C, _CHUNK * _C)], inbuf)
        b_local = ci // _CPB

        def group_body(g, carry2):
            gbase = g * (_G * _C)

            def p1(j, ms):
                x = plsc.load_gather(inbuf, [iota + (gbase + j)])
                tbuf[pl.ds(j * _G, _G)] = x
                return _insert(ms, x)

            ms = lax.fori_loop(0, _C, p1, (neg_inf,) * _K)
            m0, thr = ms[0], ms[_K - 1]

            def p2(j, zs):
                z, s = zs
                x = tbuf[pl.ds(j * _G, _G)]
                e = jnp.where(x >= thr, jnp.exp(x - m0), 0.0)
                mvec = plsc.load_gather(
                    mask_v,
                    [jnp.broadcast_to(b_local, (_G,)).astype(jnp.int32),
                     jnp.broadcast_to(j, (_G,)).astype(jnp.int32)],
                )
                t = e * mvec
                tbuf[pl.ds(j * _G, _G)] = t
                return (z + e, s + t)

            z, s = lax.fori_loop(0, _C, p2, (zero, zero))
            inv = 1.0 / (s + 1e-8 * z)

            def p3(j, c3):
                t = tbuf[pl.ds(j * _G, _G)]
                plsc.store_scatter(outbuf, [iota + (gbase + j)], t * inv)
                return c3

            lax.fori_loop(0, _C, p3, 0)
            return carry2

        lax.fori_loop(0, _GPC, group_body, 0)
        pltpu.sync_copy(outbuf, out_hbm.at[pl.ds(crow * _C, _CHUNK * _C)])
        return carry

    lax.fori_loop(0, _NCHUNK, chunk_body, 0)


def kernel(a, mask):
    a2 = a.reshape(_R * _C)
    out = _topk_mask_norm(a2, mask)
    return out.reshape(_B, _CAND, _SW, _C)


# speedup vs baseline: 7.8432x; 1.3589x over previous
import functools

import jax
import jax.numpy as jnp
from jax import lax
from jax.experimental import pallas as pl
from jax.experimental.pallas import tpu as pltpu
from jax.experimental.pallas import tpu_sc as plsc

_B = 64
_CAND = 32
_SW = 64
_C = 100
_K = 10
_R = _B * _CAND * _SW
_NC = 2
_NS = 16
_NW = _NC * _NS
_RPW = _R // _NW
_CHUNK = 128
_NCHUNK = _RPW // _CHUNK
_G = 16
_GPC = _CHUNK // _G
_RPB = _CAND * _SW
_CPB = _RPB // _CHUNK


def _insert(ms, x):
    out = [jnp.maximum(x, ms[0])]
    for i in range(1, len(ms)):
        out.append(jnp.maximum(jnp.minimum(x, ms[i - 1]), ms[i]))
    return tuple(out)


@functools.partial(
    pl.kernel,
    out_type=jax.ShapeDtypeStruct((_R * _C,), jnp.float32),
    mesh=plsc.VectorSubcoreMesh(core_axis_name="c", subcore_axis_name="s"),
    compiler_params=pltpu.CompilerParams(needs_layout_passes=False),
    scratch_types=[
        pltpu.VMEM((_CHUNK * _C,), jnp.float32),
        pltpu.VMEM((_CHUNK * _C,), jnp.float32),
        pltpu.VMEM((_G * _C,), jnp.float32),
        pltpu.VMEM((2, _C), jnp.float32),
    ],
)
def _topk_mask_norm(a_hbm, mask_hbm, out_hbm, inbuf, outbuf, tbuf, mask_v):
    wid = lax.axis_index("s") * _NC + lax.axis_index("c")
    row0 = wid * _RPW
    pltpu.sync_copy(mask_hbm.at[pl.ds(2 * wid, 2)], mask_v)
    iota = lax.iota(jnp.int32, 16) * _C
    neg_inf = jnp.full((16,), -jnp.inf, jnp.float32)
    zero = jnp.zeros((16,), jnp.float32)

    def chunk_body(ci, carry):
        crow = row0 + ci * _CHUNK
        pltpu.sync_copy(a_hbm.at[pl.ds(crow * _C, _CHUNK * _C)], inbuf)
        b_local = ci // _CPB

        def group_body(g, carry2):
            gbase = g * (_G * _C)

            def p1(j, ms):
                x = plsc.load_gather(inbuf, [iota + (gbase + j)])
                tbuf[pl.ds(j * _G, _G)] = x
                return _insert(ms, x)

            ms = lax.fori_loop(0, _C, p1, (neg_inf,) * _K, unroll=4)
            m0, thr = ms[0], ms[_K - 1]

            def p2(j, zs):
                z, s = zs
                x = tbuf[pl.ds(j * _G, _G)]
                e = jnp.where(x >= thr, jnp.exp(x - m0), 0.0)
                mvec = plsc.load_gather(
                    mask_v,
                    [jnp.broadcast_to(b_local, (_G,)).astype(jnp.int32),
                     jnp.broadcast_to(j, (_G,)).astype(jnp.int32)],
                )
                t = e * mvec
                tbuf[pl.ds(j * _G, _G)] = t
                return (z + e, s + t)

            z, s = lax.fori_loop(0, _C, p2, (zero, zero), unroll=4)
            inv = 1.0 / (s + 1e-8 * z)

            def p3(j, c3):
                t = tbuf[pl.ds(j * _G, _G)]
                plsc.store_scatter(outbuf, [iota + (gbase + j)], t * inv)
                return c3

            lax.fori_loop(0, _C, p3, 0, unroll=4)
            return carry2

        lax.fori_loop(0, _GPC, group_body, 0)
        pltpu.sync_copy(outbuf, out_hbm.at[pl.ds(crow * _C, _CHUNK * _C)])
        return carry

    lax.fori_loop(0, _NCHUNK, chunk_body, 0)


def kernel(a, mask):
    a2 = a.reshape(_R * _C)
    out = _topk_mask_norm(a2, mask)
    return out.reshape(_B, _CAND, _SW, _C)


# speedup vs baseline: 10.7094x; 1.3655x over previous
import functools

import jax
import jax.numpy as jnp
from jax import lax
from jax.experimental import pallas as pl
from jax.experimental.pallas import tpu as pltpu
from jax.experimental.pallas import tpu_sc as plsc

_B = 64
_CAND = 32
_SW = 64
_C = 100
_K = 10
_R = _B * _CAND * _SW
_NC = 2
_NS = 16
_NW = _NC * _NS
_RPW = _R // _NW
_CHUNK = 128
_NCHUNK = _RPW // _CHUNK
_G = 16
_GPC = _CHUNK // _G
_RPB = _CAND * _SW
_CPB = _RPB // _CHUNK


def _insert(ms, x):
    out = [jnp.maximum(x, ms[0])]
    for i in range(1, len(ms)):
        out.append(jnp.maximum(jnp.minimum(x, ms[i - 1]), ms[i]))
    return tuple(out)


@functools.partial(
    pl.kernel,
    out_type=jax.ShapeDtypeStruct((_R * _C,), jnp.float32),
    mesh=plsc.VectorSubcoreMesh(core_axis_name="c", subcore_axis_name="s"),
    compiler_params=pltpu.CompilerParams(needs_layout_passes=False),
    scratch_types=[
        pltpu.VMEM((_CHUNK * _C,), jnp.float32),
        pltpu.VMEM((_CHUNK * _C,), jnp.float32),
        pltpu.VMEM((_G * _C,), jnp.float32),
        pltpu.VMEM((2, _C), jnp.float32),
    ],
)
def _topk_mask_norm(a_hbm, mask_hbm, out_hbm, inbuf, outbuf, tbuf, mask_v):
    wid = lax.axis_index("s") * _NC + lax.axis_index("c")
    row0 = wid * _RPW
    pltpu.sync_copy(mask_hbm.at[pl.ds(2 * wid, 2)], mask_v)
    iota = lax.iota(jnp.int32, 16) * _C
    neg_inf = jnp.full((16,), -jnp.inf, jnp.float32)
    zero = jnp.zeros((16,), jnp.float32)

    def chunk_body(ci, carry):
        crow = row0 + ci * _CHUNK
        pltpu.sync_copy(a_hbm.at[pl.ds(crow * _C, _CHUNK * _C)], inbuf)
        b_local = ci // _CPB

        def group_body(g, carry2):
            gbase = g * (_G * _C)

            @plsc.parallel_loop(0, _C, carry=(neg_inf,) * _K, unroll=4)
            def p1(j, ms):
                x = plsc.load_gather(inbuf, [iota + (gbase + j)])
                tbuf[pl.ds(j * _G, _G)] = x
                return _insert(ms, x)

            ms = p1
            m0, thr = ms[0], ms[_K - 1]

            @plsc.parallel_loop(0, _C, carry=(zero, zero), unroll=4)
            def p2(j, zs):
                z, s = zs
                x = tbuf[pl.ds(j * _G, _G)]
                e = jnp.where(x >= thr, jnp.exp(x - m0), 0.0)
                mvec = plsc.load_gather(
                    mask_v,
                    [jnp.broadcast_to(b_local, (_G,)).astype(jnp.int32),
                     jnp.broadcast_to(j, (_G,)).astype(jnp.int32)],
                )
                t = e * mvec
                tbuf[pl.ds(j * _G, _G)] = t
                return (z + e, s + t)

            z, s = p2
            inv = 1.0 / (s + 1e-8 * z)

            @plsc.parallel_loop(0, _C, unroll=4)
            def p3(j):
                t = tbuf[pl.ds(j * _G, _G)]
                plsc.store_scatter(outbuf, [iota + (gbase + j)], t * inv)

            return carry2

        lax.fori_loop(0, _GPC, group_body, 0)
        pltpu.sync_copy(outbuf, out_hbm.at[pl.ds(crow * _C, _CHUNK * _C)])
        return carry

    lax.fori_loop(0, _NCHUNK, chunk_body, 0)


def kernel(a, mask):
    a2 = a.reshape(_R * _C)
    out = _topk_mask_norm(a2, mask)
    return out.reshape(_B, _CAND, _SW, _C)
